# baseline (speedup 1.0000x reference)
import functools

import jax
import jax.numpy as jnp
from jax import lax
from jax.experimental import pallas as pl
from jax.experimental.pallas import tpu as pltpu
from jax.experimental.pallas import tpu_sc as plsc

B = 16384
EMB_DIM = 16
N_NUMERIC = 14
TEXT_DIM = 128
PACK = 128
TAG_COL = 64
CAT_COL = 80
SIDE_COL = 96
OUT_DIM = 6 * EMB_DIM + N_NUMERIC + 64

NC = 2
NS = 16
NW = NC * NS
B_PER_W = B // NW
CHUNK = 128
N_CHUNKS = B_PER_W // CHUNK

TINY_BASE = (0, 4, 6, 8)


def _sc_body(i0, i1, i2, i3, i4, i5, tab_tag_h, tab_cat_h, out_h,
             tag_v, cat_v,
             idx_v0, idx_v1, idx_v2, idx_v3, idx_v4, idx_v5,
             pack_a, pack_b, sem):
    idx_h = (i0, i1, i2, i3, i4, i5)
    idx_v = (idx_v0, idx_v1, idx_v2, idx_v3, idx_v4, idx_v5)

    wid = lax.axis_index("s") * NC + lax.axis_index("c")
    base0 = wid * B_PER_W

    loads = [pltpu.async_copy(tab_tag_h, tag_v, sem),
             pltpu.async_copy(tab_cat_h, cat_v, sem)]
    loads += [pltpu.async_copy(idx_h[k].at[pl.ds(base0, B_PER_W)], idx_v[k],
                               sem) for k in range(6)]
    for cp in loads:
        cp.wait()

    lane = lax.iota(jnp.int32, 16)
    dst_lane = lane * PACK
    zeros16 = jnp.zeros((16,), jnp.float32)
    ones16 = jnp.ones((16,), jnp.float32)

    def do_chunk(c, buf):
        for g in range(CHUNK // 16):
            dst_g = dst_lane + g * (16 * PACK)
            for tab, k, col0 in ((tag_v, 4, TAG_COL), (cat_v, 5, CAT_COL)):
                idx16 = idx_v[k][pl.ds(c * CHUNK + g * 16, 16)]
                src = idx16 * EMB_DIM
                dst = dst_g + col0
                for col in range(EMB_DIM):
                    vals = plsc.load_gather(tab, [src + col])
                    plsc.store_scatter(buf, [dst + col], vals)
            for r in range(16):
                buf[pl.ds((g * 16 + r) * PACK + SIDE_COL, 16)] = zeros16
            for k in range(4):
                idx16 = idx_v[k][pl.ds(c * CHUNK + g * 16, 16)]
                plsc.store_scatter(
                    buf, [dst_g + (SIDE_COL + TINY_BASE[k]) + idx16], ones16)

    def pair_body(p, _):
        c0 = p * 2
        do_chunk(c0, pack_a)
        cp_a = pltpu.async_copy(
            pack_a,
            out_h.at[pl.ds((base0 + c0 * CHUNK) * PACK, CHUNK * PACK)], sem)
        do_chunk(c0 + 1, pack_b)
        cp_b = pltpu.async_copy(
            pack_b,
            out_h.at[pl.ds((base0 + (c0 + 1) * CHUNK) * PACK, CHUNK * PACK)],
            sem)
        cp_a.wait()
        cp_b.wait()
        return 0

    lax.fori_loop(0, N_CHUNKS // 2, pair_body, 0)


def _sc_gather(idxs, tab_tag, tab_cat):
    mesh = plsc.VectorSubcoreMesh(core_axis_name="c", subcore_axis_name="s")
    scratch = ([pltpu.VMEM((int(tab_tag.size),), jnp.float32),
                pltpu.VMEM((int(tab_cat.size),), jnp.float32)]
               + [pltpu.VMEM((B_PER_W,), jnp.int32)] * 6
               + [pltpu.VMEM((CHUNK * PACK,), jnp.float32)] * 2
               + [pltpu.SemaphoreType.DMA])
    k = functools.partial(
        pl.kernel, mesh=mesh,
        out_type=jax.ShapeDtypeStruct((B * PACK,), jnp.float32),
        scratch_types=scratch,
        compiler_params=pltpu.CompilerParams(use_tc_tiling_on_sc=False,
                                             needs_layout_passes=False),
    )(_sc_body)
    return k(*idxs, tab_tag.reshape(-1), tab_cat.reshape(-1)).reshape(B, PACK)


def _tc_body(packed_ref, m_ref, num_ref, cap_ref, w1_ref, b1_ref,
             w2_ref, b2_ref, g_ref, beta_ref, mean_ref, std_ref, out_ref):
    packed = packed_ref[...]
    tinye = jnp.dot(packed[:, SIDE_COL:SIDE_COL + 16], m_ref[...],
                    preferred_element_type=jnp.float32,
                    precision=lax.Precision.HIGHEST)
    num = (num_ref[...] - mean_ref[...]) / (std_ref[...] + 1e-8)
    h = jnp.dot(cap_ref[...], w1_ref[...],
                preferred_element_type=jnp.float32) + b1_ref[...]
    h = jnp.maximum(h, 0.0)
    h = jnp.dot(h, w2_ref[...], preferred_element_type=jnp.float32) + b2_ref[...]
    mu = jnp.mean(h, axis=-1, keepdims=True)
    var = jnp.mean((h - mu) * (h - mu), axis=-1, keepdims=True)
    t = (h - mu) * lax.rsqrt(var + 1e-5) * g_ref[...] + beta_ref[...]
    out_ref[...] = jnp.concatenate(
        [tinye, packed[:, TAG_COL:SIDE_COL], num, t], axis=-1)


def _tc_encode(packed, m16, numeric, caption, w1, b1, w2, b2, g, beta,
               mean, std):
    bb = 2048
    grid = (B // bb,)
    full = lambda i: (0, 0)
    row = lambda i: (i, 0)
    return pl.pallas_call(
        _tc_body,
        grid=grid,
        in_specs=[
            pl.BlockSpec((bb, PACK), row),
            pl.BlockSpec((16, 4 * EMB_DIM), full),
            pl.BlockSpec((bb, N_NUMERIC), row),
            pl.BlockSpec((bb, TEXT_DIM), row),
            pl.BlockSpec((TEXT_DIM, 128), full),
            pl.BlockSpec((1, 128), full),
            pl.BlockSpec((128, 64), full),
            pl.BlockSpec((1, 64), full),
            pl.BlockSpec((1, 64), full),
            pl.BlockSpec((1, 64), full),
            pl.BlockSpec((1, N_NUMERIC), full),
            pl.BlockSpec((1, N_NUMERIC), full),
        ],
        out_specs=pl.BlockSpec((bb, OUT_DIM), row),
        out_shape=jax.ShapeDtypeStruct((B, OUT_DIM), jnp.float32),
    )(packed, m16, numeric, caption, w1, b1.reshape(1, -1), w2,
      b2.reshape(1, -1), g.reshape(1, -1), beta.reshape(1, -1),
      mean.reshape(1, -1), std.reshape(1, -1))


def kernel(user_active_degree, is_live_streamer, is_video_author, video_type,
           tag, category_id, numeric_features, caption_embedding,
           emb_user_active_degree, emb_is_live_streamer, emb_is_video_author,
           emb_video_type, emb_tag, emb_category_id,
           W1, b1, W2, b2, ln_gamma, ln_beta, numeric_mean, numeric_std):
    idxs = [x.astype(jnp.int32) for x in
            (user_active_degree, is_live_streamer, is_video_author,
             video_type, tag, category_id)]
    packed = _sc_gather(idxs, emb_tag, emb_category_id)
    m16 = (jnp.zeros((16, 4 * EMB_DIM), jnp.float32)
           .at[0:4, 0:16].set(emb_user_active_degree)
           .at[4:6, 16:32].set(emb_is_live_streamer)
           .at[6:8, 32:48].set(emb_is_video_author)
           .at[8:10, 48:64].set(emb_video_type))
    return _tc_encode(packed, m16, numeric_features, caption_embedding,
                      W1, b1, W2, b2, ln_gamma, ln_beta, numeric_mean,
                      numeric_std)


# speedup vs baseline: 1.2388x; 1.2388x over previous
import functools

import jax
import jax.numpy as jnp
from jax import lax
from jax.experimental import pallas as pl
from jax.experimental.pallas import tpu as pltpu
from jax.experimental.pallas import tpu_sc as plsc

B = 16384
EMB_DIM = 16
N_NUMERIC = 14
TEXT_DIM = 128
PACK = 128
TAG_COL = 64
CAT_COL = 80
SIDE_COL = 96
OUT_DIM = 6 * EMB_DIM + N_NUMERIC + 64

NC = 2
NS = 16
NW = NC * NS
B_PER_W = B // NW
CHUNK = 128
N_CHUNKS = B_PER_W // CHUNK

TINY_BASE = (0, 4, 6, 8)


def _sc_body(i0, i1, i2, i3, i4, i5, tab_tag_h, tab_cat_h, out_h,
             tag_v, cat_v,
             idx_v0, idx_v1, idx_v2, idx_v3, idx_v4, idx_v5,
             pack_a, pack_b, sem):
    idx_h = (i0, i1, i2, i3, i4, i5)
    idx_v = (idx_v0, idx_v1, idx_v2, idx_v3, idx_v4, idx_v5)

    wid = lax.axis_index("s") * NC + lax.axis_index("c")
    base0 = wid * B_PER_W

    loads = [pltpu.async_copy(tab_tag_h, tag_v, sem),
             pltpu.async_copy(tab_cat_h, cat_v, sem)]
    loads += [pltpu.async_copy(idx_h[k].at[pl.ds(base0, B_PER_W)], idx_v[k],
                               sem) for k in range(6)]
    for cp in loads:
        cp.wait()

    lane = lax.iota(jnp.int32, 16)
    dst_lane = lane * PACK
    zeros16 = jnp.zeros((16,), jnp.float32)
    ones16 = jnp.ones((16,), jnp.float32)

    def do_chunk(c, buf):
        for g in range(CHUNK // 16):
            dst_g = dst_lane + g * (16 * PACK)
            for tab, k, col0 in ((tag_v, 4, TAG_COL), (cat_v, 5, CAT_COL)):
                idx16 = idx_v[k][pl.ds(c * CHUNK + g * 16, 16)]
                src = idx16 * EMB_DIM
                dst = dst_g + col0
                for col in range(EMB_DIM):
                    vals = plsc.load_gather(tab, [src + col])
                    plsc.store_scatter(buf, [dst + col], vals)
            for r in range(16):
                buf[pl.ds((g * 16 + r) * PACK + SIDE_COL, 16)] = zeros16
            for k in range(4):
                idx16 = idx_v[k][pl.ds(c * CHUNK + g * 16, 16)]
                plsc.store_scatter(
                    buf, [dst_g + (SIDE_COL + TINY_BASE[k]) + idx16], ones16)

    def pair_body(p, _):
        c0 = p * 2
        do_chunk(c0, pack_a)
        cp_a = pltpu.async_copy(
            pack_a,
            out_h.at[pl.ds((base0 + c0 * CHUNK) * PACK, CHUNK * PACK)], sem)
        do_chunk(c0 + 1, pack_b)
        cp_b = pltpu.async_copy(
            pack_b,
            out_h.at[pl.ds((base0 + (c0 + 1) * CHUNK) * PACK, CHUNK * PACK)],
            sem)
        cp_a.wait()
        cp_b.wait()
        return 0

    lax.fori_loop(0, N_CHUNKS // 2, pair_body, 0)


def _sc_gather(idxs, tab_tag, tab_cat):
    mesh = plsc.VectorSubcoreMesh(core_axis_name="c", subcore_axis_name="s")
    scratch = ([pltpu.VMEM((int(tab_tag.size),), jnp.float32),
                pltpu.VMEM((int(tab_cat.size),), jnp.float32)]
               + [pltpu.VMEM((B_PER_W,), jnp.int32)] * 6
               + [pltpu.VMEM((CHUNK * PACK,), jnp.float32)] * 2
               + [pltpu.SemaphoreType.DMA])
    k = functools.partial(
        pl.kernel, mesh=mesh,
        out_type=jax.ShapeDtypeStruct((B * PACK,), jnp.float32),
        scratch_types=scratch,
        compiler_params=pltpu.CompilerParams(use_tc_tiling_on_sc=False,
                                             needs_layout_passes=False),
    )(_sc_body)
    return k(*idxs, tab_tag.reshape(-1), tab_cat.reshape(-1)).reshape(B, PACK)


def _mlp_body(num_ref, cap_ref, w1_ref, b1_ref, w2_ref, b2_ref,
              g_ref, beta_ref, mean_ref, std_ref, out_ref):
    num = (num_ref[...] - mean_ref[...]) / (std_ref[...] + 1e-8)
    h = jnp.dot(cap_ref[...], w1_ref[...],
                preferred_element_type=jnp.float32) + b1_ref[...]
    h = jnp.maximum(h, 0.0)
    h = jnp.dot(h, w2_ref[...], preferred_element_type=jnp.float32) + b2_ref[...]
    mu = jnp.mean(h, axis=-1, keepdims=True)
    var = jnp.mean((h - mu) * (h - mu), axis=-1, keepdims=True)
    t = (h - mu) * lax.rsqrt(var + 1e-5) * g_ref[...] + beta_ref[...]
    pad = jnp.zeros((num.shape[0], PACK - N_NUMERIC - 64), jnp.float32)
    out_ref[...] = jnp.concatenate([num, t, pad], axis=-1)


def _tc_mlp(numeric, caption, w1, b1, w2, b2, g, beta, mean, std):
    bb = 2048
    grid = (B // bb,)
    full = lambda i: (0, 0)
    row = lambda i: (i, 0)
    return pl.pallas_call(
        _mlp_body,
        grid=grid,
        in_specs=[
            pl.BlockSpec((bb, N_NUMERIC), row),
            pl.BlockSpec((bb, TEXT_DIM), row),
            pl.BlockSpec((TEXT_DIM, 128), full),
            pl.BlockSpec((1, 128), full),
            pl.BlockSpec((128, 64), full),
            pl.BlockSpec((1, 64), full),
            pl.BlockSpec((1, 64), full),
            pl.BlockSpec((1, 64), full),
            pl.BlockSpec((1, N_NUMERIC), full),
            pl.BlockSpec((1, N_NUMERIC), full),
        ],
        out_specs=pl.BlockSpec((bb, PACK), row),
        out_shape=jax.ShapeDtypeStruct((B, PACK), jnp.float32),
    )(numeric, caption, w1, b1.reshape(1, -1), w2,
      b2.reshape(1, -1), g.reshape(1, -1), beta.reshape(1, -1),
      mean.reshape(1, -1), std.reshape(1, -1))


def _asm_body(packed_ref, tail_ref, m_ref, out_ref):
    packed = packed_ref[...]
    tinye = jnp.dot(packed[:, SIDE_COL:SIDE_COL + 16], m_ref[...],
                    preferred_element_type=jnp.float32,
                    precision=lax.Precision.HIGHEST)
    out_ref[...] = jnp.concatenate(
        [tinye, packed[:, TAG_COL:SIDE_COL],
         tail_ref[:, :N_NUMERIC + 64]], axis=-1)


def _tc_assemble(packed, tail, m16):
    bb = 2048
    grid = (B // bb,)
    full = lambda i: (0, 0)
    row = lambda i: (i, 0)
    return pl.pallas_call(
        _asm_body,
        grid=grid,
        in_specs=[
            pl.BlockSpec((bb, PACK), row),
            pl.BlockSpec((bb, PACK), row),
            pl.BlockSpec((16, 4 * EMB_DIM), full),
        ],
        out_specs=pl.BlockSpec((bb, OUT_DIM), row),
        out_shape=jax.ShapeDtypeStruct((B, OUT_DIM), jnp.float32),
    )(packed, tail, m16)


def kernel(user_active_degree, is_live_streamer, is_video_author, video_type,
           tag, category_id, numeric_features, caption_embedding,
           emb_user_active_degree, emb_is_live_streamer, emb_is_video_author,
           emb_video_type, emb_tag, emb_category_id,
           W1, b1, W2, b2, ln_gamma, ln_beta, numeric_mean, numeric_std):
    idxs = [x.astype(jnp.int32) for x in
            (user_active_degree, is_live_streamer, is_video_author,
             video_type, tag, category_id)]
    tail = _tc_mlp(numeric_features, caption_embedding, W1, b1, W2, b2,
                   ln_gamma, ln_beta, numeric_mean, numeric_std)
    packed = _sc_gather(idxs, emb_tag, emb_category_id)
    m16 = (jnp.zeros((16, 4 * EMB_DIM), jnp.float32)
           .at[0:4, 0:16].set(emb_user_active_degree)
           .at[4:6, 16:32].set(emb_is_live_streamer)
           .at[6:8, 32:48].set(emb_is_video_author)
           .at[8:10, 48:64].set(emb_video_type))
    return _tc_assemble(packed, tail, m16)


# speedup vs baseline: 1.5546x; 1.2549x over previous
import functools

import jax
import jax.numpy as jnp
from jax import lax
from jax.experimental import pallas as pl
from jax.experimental.pallas import tpu as pltpu
from jax.experimental.pallas import tpu_sc as plsc

B = 16384
EMB_DIM = 16
N_NUMERIC = 14
TEXT_DIM = 128
PACK = 128
TAG_ROW = 64
CAT_ROW = 80
SIDE_ROW = 96
TAIL_ROWS = N_NUMERIC + 64
OUT_DIM = 6 * EMB_DIM + TAIL_ROWS

NC = 2
NS = 16
NW = NC * NS
B_PER_W = B // NW
CHUNK = 128
N_CHUNKS = B_PER_W // CHUNK

TINY_BASE = (0, 4, 6, 8)


def _sc_body(i0, i1, i2, i3, i4, i5, tab_tag_h, tab_cat_h, out_h,
             tag_v, cat_v,
             idx_v0, idx_v1, idx_v2, idx_v3, idx_v4, idx_v5,
             pack_a, pack_b, sem):
    idx_h = (i0, i1, i2, i3, i4, i5)
    idx_v = (idx_v0, idx_v1, idx_v2, idx_v3, idx_v4, idx_v5)

    wid = lax.axis_index("s") * NC + lax.axis_index("c")
    base0 = wid * B_PER_W

    loads = [pltpu.async_copy(tab_tag_h, tag_v, sem),
             pltpu.async_copy(tab_cat_h, cat_v, sem)]
    loads += [pltpu.async_copy(idx_h[k].at[pl.ds(base0, B_PER_W)], idx_v[k],
                               sem) for k in range(6)]
    for cp in loads:
        cp.wait()

    lane = lax.iota(jnp.int32, 16)
    zeros16 = jnp.zeros((16,), jnp.float32)
    ones16 = jnp.ones((16,), jnp.float32)

    def do_chunk(c, buf):
        for r in range(16):
            for j in range(CHUNK // 16):
                buf[SIDE_ROW + r, pl.ds(j * 16, 16)] = zeros16
        for g in range(CHUNK // 16):
            for tab, k, row0 in ((tag_v, 4, TAG_ROW), (cat_v, 5, CAT_ROW)):
                idx16 = idx_v[k][pl.ds(c * CHUNK + g * 16, 16)]
                src = idx16 * EMB_DIM
                for col in range(EMB_DIM):
                    vals = plsc.load_gather(tab, [src + col])
                    buf[row0 + col, pl.ds(g * 16, 16)] = vals
            for k in range(4):
                idx16 = idx_v[k][pl.ds(c * CHUNK + g * 16, 16)]
                plsc.store_scatter(
                    buf,
                    [idx16 + (SIDE_ROW + TINY_BASE[k]), lane + g * 16],
                    ones16)

    def pair_body(p, _):
        c0 = p * 2
        do_chunk(c0, pack_a)
        cp_a = pltpu.async_copy(
            pack_a, out_h.at[:, pl.ds(base0 + c0 * CHUNK, CHUNK)], sem)
        do_chunk(c0 + 1, pack_b)
        cp_b = pltpu.async_copy(
            pack_b, out_h.at[:, pl.ds(base0 + (c0 + 1) * CHUNK, CHUNK)], sem)
        cp_a.wait()
        cp_b.wait()
        return 0

    lax.fori_loop(0, N_CHUNKS // 2, pair_body, 0)


def _sc_gather(idxs, tab_tag, tab_cat):
    mesh = plsc.VectorSubcoreMesh(core_axis_name="c", subcore_axis_name="s")
    scratch = ([pltpu.VMEM((int(tab_tag.size),), jnp.float32),
                pltpu.VMEM((int(tab_cat.size),), jnp.float32)]
               + [pltpu.VMEM((B_PER_W,), jnp.int32)] * 6
               + [pltpu.VMEM((PACK, CHUNK), jnp.float32)] * 2
               + [pltpu.SemaphoreType.DMA])
    k = functools.partial(
        pl.kernel, mesh=mesh,
        out_type=jax.ShapeDtypeStruct((PACK, B), jnp.float32),
        scratch_types=scratch,
        compiler_params=pltpu.CompilerParams(use_tc_tiling_on_sc=False,
                                             needs_layout_passes=False),
    )(_sc_body)
    return k(*idxs, tab_tag.reshape(-1), tab_cat.reshape(-1))


def _mlp_body(numt_ref, cap_ref, w1_ref, b1_ref, w2_ref, b2_ref,
              g_ref, beta_ref, mean_ref, std_ref, out_ref):
    num = (numt_ref[...] - mean_ref[...]) / (std_ref[...] + 1e-8)
    h = lax.dot_general(w1_ref[...], cap_ref[...], (((0,), (1,)), ((), ())),
                        preferred_element_type=jnp.float32)
    h = jnp.maximum(h + b1_ref[...], 0.0)
    h = lax.dot_general(w2_ref[...], h, (((0,), (0,)), ((), ())),
                        preferred_element_type=jnp.float32) + b2_ref[...]
    mu = jnp.mean(h, axis=0, keepdims=True)
    var = jnp.mean((h - mu) * (h - mu), axis=0, keepdims=True)
    t = (h - mu) * lax.rsqrt(var + 1e-5) * g_ref[...] + beta_ref[...]
    out_ref[...] = jnp.concatenate([num, t], axis=0)


def _tc_mlp(numeric_t, caption, w1, b1, w2, b2, g, beta, mean, std):
    bb = 2048
    grid = (B // bb,)
    full = lambda i: (0, 0)
    return pl.pallas_call(
        _mlp_body,
        grid=grid,
        in_specs=[
            pl.BlockSpec((N_NUMERIC, bb), lambda i: (0, i)),
            pl.BlockSpec((bb, TEXT_DIM), lambda i: (i, 0)),
            pl.BlockSpec((TEXT_DIM, 128), full),
            pl.BlockSpec((128, 1), full),
            pl.BlockSpec((128, 64), full),
            pl.BlockSpec((64, 1), full),
            pl.BlockSpec((64, 1), full),
            pl.BlockSpec((64, 1), full),
            pl.BlockSpec((N_NUMERIC, 1), full),
            pl.BlockSpec((N_NUMERIC, 1), full),
        ],
        out_specs=pl.BlockSpec((TAIL_ROWS, bb), lambda i: (0, i)),
        out_shape=jax.ShapeDtypeStruct((TAIL_ROWS, B), jnp.float32),
    )(numeric_t, caption, w1, b1.reshape(-1, 1), w2,
      b2.reshape(-1, 1), g.reshape(-1, 1), beta.reshape(-1, 1),
      mean.reshape(-1, 1), std.reshape(-1, 1))


def _asm_body(packed_ref, tail_ref, m_ref, out_ref):
    packed = packed_ref[...]
    tinye = lax.dot_general(m_ref[...], packed[SIDE_ROW:SIDE_ROW + 16, :],
                            (((0,), (0,)), ((), ())),
                            preferred_element_type=jnp.float32,
                            precision=lax.Precision.HIGHEST)
    out_ref[...] = jnp.concatenate(
        [tinye, packed[TAG_ROW:SIDE_ROW, :], tail_ref[...]], axis=0)


def _tc_assemble(packed, tail, m16):
    bb = 2048
    grid = (B // bb,)
    return pl.pallas_call(
        _asm_body,
        grid=grid,
        in_specs=[
            pl.BlockSpec((PACK, bb), lambda i: (0, i)),
            pl.BlockSpec((TAIL_ROWS, bb), lambda i: (0, i)),
            pl.BlockSpec((16, 4 * EMB_DIM), lambda i: (0, 0)),
        ],
        out_specs=pl.BlockSpec((OUT_DIM, bb), lambda i: (0, i)),
        out_shape=jax.ShapeDtypeStruct((OUT_DIM, B), jnp.float32),
    )(packed, tail, m16)


def kernel(user_active_degree, is_live_streamer, is_video_author, video_type,
           tag, category_id, numeric_features, caption_embedding,
           emb_user_active_degree, emb_is_live_streamer, emb_is_video_author,
           emb_video_type, emb_tag, emb_category_id,
           W1, b1, W2, b2, ln_gamma, ln_beta, numeric_mean, numeric_std):
    idxs = [x.astype(jnp.int32) for x in
            (user_active_degree, is_live_streamer, is_video_author,
             video_type, tag, category_id)]
    tail = _tc_mlp(numeric_features.T, caption_embedding, W1, b1, W2, b2,
                   ln_gamma, ln_beta, numeric_mean, numeric_std)
    packed = _sc_gather(idxs, emb_tag, emb_category_id)
    m16 = (jnp.zeros((16, 4 * EMB_DIM), jnp.float32)
           .at[0:4, 0:16].set(emb_user_active_degree)
           .at[4:6, 16:32].set(emb_is_live_streamer)
           .at[6:8, 32:48].set(emb_is_video_author)
           .at[8:10, 48:64].set(emb_video_type))
    return _tc_assemble(packed, tail, m16).T


# speedup vs baseline: 1.8556x; 1.1937x over previous
import functools

import jax
import jax.numpy as jnp
from jax import lax
from jax.experimental import pallas as pl
from jax.experimental.pallas import tpu as pltpu
from jax.experimental.pallas import tpu_sc as plsc

B = 16384
EMB_DIM = 16
N_NUMERIC = 14
TEXT_DIM = 128
PACK = 128
TAG_ROW = 64
CAT_ROW = 80
SIDE_ROW = 96
TAIL_ROWS = N_NUMERIC + 64
OUT_DIM = 6 * EMB_DIM + TAIL_ROWS

NC = 2
NS = 16
NW = NC * NS
B_PER_W = B // NW
CHUNK = 128
N_CHUNKS = B_PER_W // CHUNK

TINY_BASE = (0, 4, 6, 8)


def _sc_body(i0, i1, i2, i3, i4, i5, tab_tag_h, tab_cat_h, out_h,
             tag_v, cat_v,
             idx_v0, idx_v1, idx_v2, idx_v3, idx_v4, idx_v5,
             pack_a, pack_b, sem):
    idx_h = (i0, i1, i2, i3, i4, i5)
    idx_v = (idx_v0, idx_v1, idx_v2, idx_v3, idx_v4, idx_v5)

    wid = lax.axis_index("s") * NC + lax.axis_index("c")
    base0 = wid * B_PER_W

    loads = [pltpu.async_copy(tab_tag_h, tag_v, sem),
             pltpu.async_copy(tab_cat_h, cat_v, sem)]
    loads += [pltpu.async_copy(idx_h[k].at[pl.ds(base0, B_PER_W)], idx_v[k],
                               sem) for k in range(6)]
    for cp in loads:
        cp.wait()

    lane = lax.iota(jnp.int32, 16)
    zeros16 = jnp.zeros((16,), jnp.float32)
    ones16 = jnp.ones((16,), jnp.float32)

    def do_chunk(c, buf):
        for r in range(16):
            for j in range(CHUNK // 16):
                buf[SIDE_ROW + r, pl.ds(j * 16, 16)] = zeros16
        for g in range(CHUNK // 16):
            for tab, k, row0, voc in ((tag_v, 4, TAG_ROW, 1000),
                                      (cat_v, 5, CAT_ROW, 100)):
                idx16 = idx_v[k][pl.ds(c * CHUNK + g * 16, 16)]
                for col in range(EMB_DIM):
                    vals = plsc.load_gather(tab, [idx16 + col * voc])
                    buf[row0 + col, pl.ds(g * 16, 16)] = vals
            for k in range(4):
                idx16 = idx_v[k][pl.ds(c * CHUNK + g * 16, 16)]
                plsc.store_scatter(
                    buf,
                    [idx16 + (SIDE_ROW + TINY_BASE[k]), lane + g * 16],
                    ones16)

    chunk0 = base0 // CHUNK

    def pair_body(p, _):
        c0 = p * 2
        do_chunk(c0, pack_a)
        cp_a = pltpu.async_copy(
            pack_a, out_h.at[chunk0 + c0], sem)
        do_chunk(c0 + 1, pack_b)
        cp_b = pltpu.async_copy(
            pack_b, out_h.at[chunk0 + c0 + 1], sem)
        cp_a.wait()
        cp_b.wait()
        return 0

    lax.fori_loop(0, N_CHUNKS // 2, pair_body, 0)


def _sc_gather(idxs, tab_tag, tab_cat):
    mesh = plsc.VectorSubcoreMesh(core_axis_name="c", subcore_axis_name="s")
    scratch = ([pltpu.VMEM((int(tab_tag.size),), jnp.float32),
                pltpu.VMEM((int(tab_cat.size),), jnp.float32)]
               + [pltpu.VMEM((B_PER_W,), jnp.int32)] * 6
               + [pltpu.VMEM((PACK, CHUNK), jnp.float32)] * 2
               + [pltpu.SemaphoreType.DMA])
    k = functools.partial(
        pl.kernel, mesh=mesh,
        out_type=jax.ShapeDtypeStruct((B // CHUNK, PACK, CHUNK), jnp.float32),
        scratch_types=scratch,
        compiler_params=pltpu.CompilerParams(use_tc_tiling_on_sc=False,
                                             needs_layout_passes=False),
    )(_sc_body)
    return k(*idxs, tab_tag.T.reshape(-1), tab_cat.T.reshape(-1))


def _mlp_body(numt_ref, cap_ref, w1_ref, b1_ref, w2_ref, b2_ref,
              g_ref, beta_ref, mean_ref, std_ref, out_ref):
    num = (numt_ref[...] - mean_ref[...]) / (std_ref[...] + 1e-8)
    h = lax.dot_general(w1_ref[...], cap_ref[...], (((0,), (1,)), ((), ())),
                        preferred_element_type=jnp.float32)
    h = jnp.maximum(h + b1_ref[...], 0.0)
    h = lax.dot_general(w2_ref[...], h, (((0,), (0,)), ((), ())),
                        preferred_element_type=jnp.float32) + b2_ref[...]
    mu = jnp.mean(h, axis=0, keepdims=True)
    var = jnp.mean((h - mu) * (h - mu), axis=0, keepdims=True)
    t = (h - mu) * lax.rsqrt(var + 1e-5) * g_ref[...] + beta_ref[...]
    out_ref[...] = jnp.concatenate([num, t], axis=0)


def _tc_mlp(numeric_t, caption, w1, b1, w2, b2, g, beta, mean, std):
    bb = 2048
    grid = (B // bb,)
    full = lambda i: (0, 0)
    return pl.pallas_call(
        _mlp_body,
        grid=grid,
        in_specs=[
            pl.BlockSpec((N_NUMERIC, bb), lambda i: (0, i)),
            pl.BlockSpec((bb, TEXT_DIM), lambda i: (i, 0)),
            pl.BlockSpec((TEXT_DIM, 128), full),
            pl.BlockSpec((128, 1), full),
            pl.BlockSpec((128, 64), full),
            pl.BlockSpec((64, 1), full),
            pl.BlockSpec((64, 1), full),
            pl.BlockSpec((64, 1), full),
            pl.BlockSpec((N_NUMERIC, 1), full),
            pl.BlockSpec((N_NUMERIC, 1), full),
        ],
        out_specs=pl.BlockSpec((TAIL_ROWS, bb), lambda i: (0, i)),
        out_shape=jax.ShapeDtypeStruct((TAIL_ROWS, B), jnp.float32),
    )(numeric_t, caption, w1, b1.reshape(-1, 1), w2,
      b2.reshape(-1, 1), g.reshape(-1, 1), beta.reshape(-1, 1),
      mean.reshape(-1, 1), std.reshape(-1, 1))


def _asm_body(packed_ref, tail_ref, m_ref, out_ref):
    slabs = packed_ref[...]
    packed = jnp.concatenate(
        [slabs[j] for j in range(slabs.shape[0])], axis=-1)
    tinye = lax.dot_general(m_ref[...], packed[SIDE_ROW:SIDE_ROW + 16, :],
                            (((0,), (0,)), ((), ())),
                            preferred_element_type=jnp.float32,
                            precision=lax.Precision.HIGHEST)
    out_ref[...] = jnp.concatenate(
        [tinye, packed[TAG_ROW:SIDE_ROW, :], tail_ref[...]], axis=0)


def _tc_assemble(packed3, tail, m16):
    bb = 2048
    grid = (B // bb,)
    return pl.pallas_call(
        _asm_body,
        grid=grid,
        in_specs=[
            pl.BlockSpec((bb // CHUNK, PACK, CHUNK), lambda i: (i, 0, 0)),
            pl.BlockSpec((TAIL_ROWS, bb), lambda i: (0, i)),
            pl.BlockSpec((16, 4 * EMB_DIM), lambda i: (0, 0)),
        ],
        out_specs=pl.BlockSpec((OUT_DIM, bb), lambda i: (0, i)),
        out_shape=jax.ShapeDtypeStruct((OUT_DIM, B), jnp.float32),
    )(packed3, tail, m16)


def kernel(user_active_degree, is_live_streamer, is_video_author, video_type,
           tag, category_id, numeric_features, caption_embedding,
           emb_user_active_degree, emb_is_live_streamer, emb_is_video_author,
           emb_video_type, emb_tag, emb_category_id,
           W1, b1, W2, b2, ln_gamma, ln_beta, numeric_mean, numeric_std):
    idxs = [x.astype(jnp.int32) for x in
            (user_active_degree, is_live_streamer, is_video_author,
             video_type, tag, category_id)]
    tail = _tc_mlp(numeric_features.T, caption_embedding, W1, b1, W2, b2,
                   ln_gamma, ln_beta, numeric_mean, numeric_std)
    packed = _sc_gather(idxs, emb_tag, emb_category_id)
    m16 = (jnp.zeros((16, 4 * EMB_DIM), jnp.float32)
           .at[0:4, 0:16].set(emb_user_active_degree)
           .at[4:6, 16:32].set(emb_is_live_streamer)
           .at[6:8, 32:48].set(emb_is_video_author)
           .at[8:10, 48:64].set(emb_video_type))
    return _tc_assemble(packed, tail, m16).T


# speedup vs baseline: 1.9484x; 1.0500x over previous
import functools

import jax
import jax.numpy as jnp
from jax import lax
from jax.experimental import pallas as pl
from jax.experimental.pallas import tpu as pltpu
from jax.experimental.pallas import tpu_sc as plsc

B = 16384
EMB_DIM = 16
N_NUMERIC = 14
TEXT_DIM = 128
PACK = 48
TAG_ROW = 0
CAT_ROW = 16
SIDE_ROW = 32
TAIL_ROWS = N_NUMERIC + 64
OUT_DIM = 6 * EMB_DIM + TAIL_ROWS

NC = 2
NS = 16
NW = NC * NS
B_PER_W = B // NW
CHUNK = 128
N_CHUNKS = B_PER_W // CHUNK

TINY_BASE = (0, 4, 6, 8)


def _sc_body(i0, i1, i2, i3, i4, i5, tab_tag_h, tab_cat_h, out_h,
             tag_v, cat_v,
             idx_v0, idx_v1, idx_v2, idx_v3, idx_v4, idx_v5,
             pack_a, pack_b, sem):
    idx_h = (i0, i1, i2, i3, i4, i5)
    idx_v = (idx_v0, idx_v1, idx_v2, idx_v3, idx_v4, idx_v5)

    wid = lax.axis_index("s") * NC + lax.axis_index("c")
    base0 = wid * B_PER_W

    loads = [pltpu.async_copy(tab_tag_h, tag_v, sem),
             pltpu.async_copy(tab_cat_h, cat_v, sem)]
    loads += [pltpu.async_copy(idx_h[k].at[pl.ds(base0, B_PER_W)], idx_v[k],
                               sem) for k in range(6)]
    for cp in loads:
        cp.wait()

    lane = lax.iota(jnp.int32, 16)
    zeros16 = jnp.zeros((16,), jnp.float32)
    ones16 = jnp.ones((16,), jnp.float32)
    zero_i = jnp.zeros((16,), jnp.int32)
    col_splat = [zero_i + col for col in range(EMB_DIM)]

    def do_chunk(c, buf):
        for r in range(16):
            for j in range(CHUNK // 16):
                buf[SIDE_ROW + r, pl.ds(j * 16, 16)] = zeros16
        for g in range(CHUNK // 16):
            for tab, k, row0 in ((tag_v, 4, TAG_ROW), (cat_v, 5, CAT_ROW)):
                idx16 = idx_v[k][pl.ds(c * CHUNK + g * 16, 16)]
                for col in range(EMB_DIM):
                    vals = plsc.load_gather(tab, [col_splat[col], idx16])
                    buf[row0 + col, pl.ds(g * 16, 16)] = vals
            for k in range(4):
                idx16 = idx_v[k][pl.ds(c * CHUNK + g * 16, 16)]
                plsc.store_scatter(
                    buf,
                    [idx16 + (SIDE_ROW + TINY_BASE[k]), lane + g * 16],
                    ones16)

    chunk0 = base0 // CHUNK

    def pair_body(p, _):
        c0 = p * 2
        do_chunk(c0, pack_a)
        cp_a = pltpu.async_copy(
            pack_a, out_h.at[chunk0 + c0], sem)
        do_chunk(c0 + 1, pack_b)
        cp_b = pltpu.async_copy(
            pack_b, out_h.at[chunk0 + c0 + 1], sem)
        cp_a.wait()
        cp_b.wait()
        return 0

    lax.fori_loop(0, N_CHUNKS // 2, pair_body, 0)


def _sc_gather(idxs, tab_tag, tab_cat):
    mesh = plsc.VectorSubcoreMesh(core_axis_name="c", subcore_axis_name="s")
    scratch = ([pltpu.VMEM((EMB_DIM, tab_tag.shape[0]), jnp.float32),
                pltpu.VMEM((EMB_DIM, tab_cat.shape[0]), jnp.float32)]
               + [pltpu.VMEM((B_PER_W,), jnp.int32)] * 6
               + [pltpu.VMEM((PACK, CHUNK), jnp.float32)] * 2
               + [pltpu.SemaphoreType.DMA])
    k = functools.partial(
        pl.kernel, mesh=mesh,
        out_type=jax.ShapeDtypeStruct((B // CHUNK, PACK, CHUNK), jnp.float32),
        scratch_types=scratch,
        compiler_params=pltpu.CompilerParams(use_tc_tiling_on_sc=False,
                                             needs_layout_passes=False),
    )(_sc_body)
    return k(*idxs, tab_tag.T, tab_cat.T)


def _mlp_body(numt_ref, cap_ref, w1_ref, b1_ref, w2_ref, b2_ref,
              g_ref, beta_ref, mean_ref, std_ref, out_ref):
    num = (numt_ref[...] - mean_ref[...]) / (std_ref[...] + 1e-8)
    h = lax.dot_general(w1_ref[...], cap_ref[...], (((0,), (1,)), ((), ())),
                        preferred_element_type=jnp.float32)
    h = jnp.maximum(h + b1_ref[...], 0.0)
    h = lax.dot_general(w2_ref[...], h, (((0,), (0,)), ((), ())),
                        preferred_element_type=jnp.float32) + b2_ref[...]
    mu = jnp.mean(h, axis=0, keepdims=True)
    var = jnp.mean((h - mu) * (h - mu), axis=0, keepdims=True)
    t = (h - mu) * lax.rsqrt(var + 1e-5) * g_ref[...] + beta_ref[...]
    out_ref[...] = jnp.concatenate([num, t], axis=0)


def _tc_mlp(numeric_t, caption, w1, b1, w2, b2, g, beta, mean, std):
    bb = 2048
    grid = (B // bb,)
    full = lambda i: (0, 0)
    return pl.pallas_call(
        _mlp_body,
        grid=grid,
        in_specs=[
            pl.BlockSpec((N_NUMERIC, bb), lambda i: (0, i)),
            pl.BlockSpec((bb, TEXT_DIM), lambda i: (i, 0)),
            pl.BlockSpec((TEXT_DIM, 128), full),
            pl.BlockSpec((128, 1), full),
            pl.BlockSpec((128, 64), full),
            pl.BlockSpec((64, 1), full),
            pl.BlockSpec((64, 1), full),
            pl.BlockSpec((64, 1), full),
            pl.BlockSpec((N_NUMERIC, 1), full),
            pl.BlockSpec((N_NUMERIC, 1), full),
        ],
        out_specs=pl.BlockSpec((TAIL_ROWS, bb), lambda i: (0, i)),
        out_shape=jax.ShapeDtypeStruct((TAIL_ROWS, B), jnp.float32),
    )(numeric_t, caption, w1, b1.reshape(-1, 1), w2,
      b2.reshape(-1, 1), g.reshape(-1, 1), beta.reshape(-1, 1),
      mean.reshape(-1, 1), std.reshape(-1, 1))


def _asm_body(packed_ref, tail_ref, m_ref, out_ref):
    slabs = packed_ref[...]
    packed = jnp.concatenate(
        [slabs[j] for j in range(slabs.shape[0])], axis=-1)
    tinye = lax.dot_general(m_ref[...], packed[SIDE_ROW:SIDE_ROW + 16, :],
                            (((0,), (0,)), ((), ())),
                            preferred_element_type=jnp.float32,
                            precision=lax.Precision.HIGHEST)
    out_ref[...] = jnp.concatenate(
        [tinye, packed[TAG_ROW:SIDE_ROW, :], tail_ref[...]], axis=0)


def _tc_assemble(packed3, tail, m16):
    bb = 2048
    grid = (B // bb,)
    return pl.pallas_call(
        _asm_body,
        grid=grid,
        in_specs=[
            pl.BlockSpec((bb // CHUNK, PACK, CHUNK), lambda i: (i, 0, 0)),
            pl.BlockSpec((TAIL_ROWS, bb), lambda i: (0, i)),
            pl.BlockSpec((16, 4 * EMB_DIM), lambda i: (0, 0)),
        ],
        out_specs=pl.BlockSpec((OUT_DIM, bb), lambda i: (0, i)),
        out_shape=jax.ShapeDtypeStruct((OUT_DIM, B), jnp.float32),
    )(packed3, tail, m16)


def kernel(user_active_degree, is_live_streamer, is_video_author, video_type,
           tag, category_id, numeric_features, caption_embedding,
           emb_user_active_degree, emb_is_live_streamer, emb_is_video_author,
           emb_video_type, emb_tag, emb_category_id,
           W1, b1, W2, b2, ln_gamma, ln_beta, numeric_mean, numeric_std):
    idxs = [x.astype(jnp.int32) for x in
            (user_active_degree, is_live_streamer, is_video_author,
             video_type, tag, category_id)]
    tail = _tc_mlp(numeric_features.T, caption_embedding, W1, b1, W2, b2,
                   ln_gamma, ln_beta, numeric_mean, numeric_std)
    packed = _sc_gather(idxs, emb_tag, emb_category_id)
    m16 = (jnp.zeros((16, 4 * EMB_DIM), jnp.float32)
           .at[0:4, 0:16].set(emb_user_active_degree)
           .at[4:6, 16:32].set(emb_is_live_streamer)
           .at[6:8, 32:48].set(emb_is_video_author)
           .at[8:10, 48:64].set(emb_video_type))
    return _tc_assemble(packed, tail, m16).T
